# baseline (speedup 1.0000x reference)
import functools

import jax
import jax.numpy as jnp
from jax.experimental import pallas as pl

N_NODES = 10000
N_EDGES = 320000
D = 128
N_GRAPHS = 200
K = 30
KS = 5


def _head_body(flat_ref, w1_ref, b1_ref, w2_ref, b2_ref, o_ref):
    hid = jnp.maximum(flat_ref[...] @ w1_ref[...] + b1_ref[...][None, :], 0.0)
    o_ref[...] = hid @ w2_ref[...] + b2_ref[...][None, :]


def _head(flat, d1_w, d1_b, d2_w, d2_b):
    return pl.pallas_call(
        _head_body,
        out_shape=jax.ShapeDtypeStruct((flat.shape[0], d2_w.shape[1]), flat.dtype),
    )(flat, d1_w, d1_b, d2_w, d2_b)


def _gcn(h, row, col, norm, W, b):
    m = h @ W
    msg = m[row] * norm[:, None]
    out = jax.ops.segment_sum(msg, col, num_segments=N_NODES)
    return jax.nn.relu(out + b)


def kernel(x, edge_index, edge_weight, node_graph_index, W1, b1, W2, b2, W3, b3,
           conv_w, conv_b, d1_w, d1_b, d2_w, d2_b):
    loop = jnp.arange(N_NODES, dtype=edge_index.dtype)
    row = jnp.concatenate([edge_index[0], loop])
    col = jnp.concatenate([edge_index[1], loop])
    w = jnp.concatenate([edge_weight, jnp.ones((N_NODES,), dtype=edge_weight.dtype)])
    deg = jax.ops.segment_sum(w, row, num_segments=N_NODES)
    dis = deg ** -0.5
    norm = dis[row] * w * dis[col]
    h = _gcn(x, row, col, norm, W1, b1)
    h = _gcn(h, row, col, norm, W2, b2)
    h = _gcn(h, row, col, norm, W3, b3)
    score = h[:, -1]
    order = jnp.lexsort((-score, node_graph_index))
    g_sorted = node_graph_index[order]
    counts = jnp.bincount(node_graph_index, length=N_GRAPHS)
    starts = jnp.cumsum(counts) - counts
    pos = jnp.arange(N_NODES, dtype=jnp.int32) - starts[g_sorted].astype(jnp.int32)
    mask = (pos < K).astype(h.dtype)
    h_sorted = h[order] * mask[:, None]
    pos_c = jnp.minimum(pos, K - 1)
    pooled = jnp.zeros((N_GRAPHS, K, h.shape[-1]), dtype=h.dtype).at[g_sorted, pos_c].add(h_sorted)
    conv = jax.lax.conv_general_dilated(
        pooled, conv_w, window_strides=(1,), padding='VALID',
        dimension_numbers=('NWC', 'WIO', 'NWC'))
    conv = jax.nn.relu(conv + conv_b)
    flat = conv.reshape(N_GRAPHS, -1)
    return _head(flat, d1_w, d1_b, d2_w, d2_b)


# speedup vs baseline: 4.6195x; 4.6195x over previous
import dataclasses
import functools

import jax
import jax.numpy as jnp
from jax import lax
from jax.experimental import pallas as pl
from jax.experimental.pallas import tpu as pltpu
from jax.experimental.pallas import tpu_sc as plsc

N_NODES = 10000
N_EDGES = 320000
D = 128
N_GRAPHS = 200
K = 30
KS = 5

NC = 2
NS = 16
CHUNK = 128
CHUNKS = 80
EPT = CHUNK * CHUNKS
E_PAD = EPT * NC * NS
SBLK = 80
NSBLK = N_NODES // SBLK
DH = D // 2

_mesh = plsc.VectorSubcoreMesh(core_axis_name="c", subcore_axis_name="s")

_sc_params = pltpu.CompilerParams()
for _f, _v in (("needs_layout_passes", False), ("use_tc_tiling_on_sc", False)):
    if _f in pltpu.CompilerParams.__dataclass_fields__:
        _sc_params = dataclasses.replace(_sc_params, **{_f: _v})


def _msg_body(mp0_hbm, mp1_hbm, row_hbm, col_hbm, w_hbm, out_hbm,
              row_v, col_v, w_v, gbuf, zbuf, acc, sem):
    c = lax.axis_index("c")
    s = lax.axis_index("s")
    pltpu.sync_copy(row_hbm.at[c, s], row_v)
    pltpu.sync_copy(col_hbm.at[c, s], col_v)
    pltpu.sync_copy(w_hbm.at[c, s], w_v)

    @pl.loop(0, SBLK)
    def _(i):
        for cc in range(DH // 16):
            zbuf[i, pl.ds(cc * 16, 16)] = jnp.zeros((16,), jnp.float32)

    for hh, mp_hbm in enumerate((mp0_hbm, mp1_hbm)):
        for k in range((NSBLK + NS - 1) // NS):
            blk = s + k * NS

            @pl.when(blk < NSBLK)
            def _():
                pltpu.sync_copy(zbuf, acc.at[pl.ds(blk * SBLK, SBLK)])
        plsc.subcore_barrier()

        @pl.loop(0, CHUNKS)
        def _(j):
            pltpu.async_copy(mp_hbm.at[row_v.at[j]], gbuf, sem).wait()

            @pl.loop(0, CHUNK)
            def _(e):
                wv = plsc.load_gather(
                    w_v,
                    [jnp.full((16,), j, jnp.int32), jnp.full((16,), e, jnp.int32)])
                for cc in range(DH // 16):
                    gbuf[e, pl.ds(cc * 16, 16)] = gbuf[e, pl.ds(cc * 16, 16)] * wv

            pltpu.sync_copy(gbuf, acc.at[col_v.at[j]], add=True)

        plsc.subcore_barrier()
        for k in range((NSBLK + NS - 1) // NS):
            blk = s + k * NS

            @pl.when(blk < NSBLK)
            def _():
                off = blk * SBLK
                pltpu.sync_copy(acc.at[pl.ds(off, SBLK)],
                                out_hbm.at[c, hh, pl.ds(off, SBLK)])
        plsc.subcore_barrier()


@functools.partial(
    pl.kernel,
    out_type=jax.ShapeDtypeStruct((NC, 2, N_NODES, DH), jnp.float32),
    mesh=_mesh,
    scratch_types=[
        pltpu.VMEM((CHUNKS, CHUNK), jnp.int32),
        pltpu.VMEM((CHUNKS, CHUNK), jnp.int32),
        pltpu.VMEM((CHUNKS, CHUNK), jnp.float32),
        pltpu.VMEM((CHUNK, DH), jnp.float32),
        pltpu.VMEM((SBLK, DH), jnp.float32),
        pltpu.VMEM_SHARED((N_NODES, DH), jnp.float32),
        pltpu.SemaphoreType.DMA,
    ],
    compiler_params=_sc_params,
)
def _sc_msg(*args):
    _msg_body(*args)


def _head_body(flat_ref, w1_ref, b1_ref, w2_ref, b2_ref, o_ref):
    hid = jnp.maximum(flat_ref[...] @ w1_ref[...] + b1_ref[...][None, :], 0.0)
    o_ref[...] = hid @ w2_ref[...] + b2_ref[...][None, :]


def _head(flat, d1_w, d1_b, d2_w, d2_b):
    return pl.pallas_call(
        _head_body,
        out_shape=jax.ShapeDtypeStruct((flat.shape[0], d2_w.shape[1]), flat.dtype),
    )(flat, d1_w, d1_b, d2_w, d2_b)


def kernel(x, edge_index, edge_weight, node_graph_index, W1, b1, W2, b2, W3, b3,
           conv_w, conv_b, d1_w, d1_b, d2_w, d2_b):
    row = edge_index[0]
    col = edge_index[1]
    w = edge_weight
    deg = jax.ops.segment_sum(w, row, num_segments=N_NODES) + 1.0
    dis = deg ** -0.5
    dis_col = dis[:, None]
    dis2 = (dis * dis)[:, None]

    pad = E_PAD - N_EDGES
    row_a = jnp.pad(row, (0, pad)).reshape(NC, NS, CHUNKS, CHUNK)
    col_a = jnp.pad(col, (0, pad)).reshape(NC, NS, CHUNKS, CHUNK)
    w_a = jnp.pad(w, (0, pad)).reshape(NC, NS, CHUNKS, CHUNK)

    def gcn(h, W, b):
        m = h @ W
        mp = m * dis_col
        acc = _sc_msg(mp[:, :DH], mp[:, DH:], row_a, col_a, w_a)
        accsum = acc[0] + acc[1]
        agg = dis_col * jnp.concatenate([accsum[0], accsum[1]], axis=1) + dis2 * m
        return jax.nn.relu(agg + b)

    h = gcn(x, W1, b1)
    h = gcn(h, W2, b2)
    h = gcn(h, W3, b3)

    score = h[:, -1]
    order = jnp.lexsort((-score, node_graph_index))
    g_sorted = node_graph_index[order]
    counts = jnp.bincount(node_graph_index, length=N_GRAPHS)
    starts = jnp.cumsum(counts) - counts
    pos = jnp.arange(N_NODES, dtype=jnp.int32) - starts[g_sorted].astype(jnp.int32)
    mask = (pos < K).astype(h.dtype)
    h_sorted = h[order] * mask[:, None]
    pos_c = jnp.minimum(pos, K - 1)
    pooled = jnp.zeros((N_GRAPHS, K, D), dtype=h.dtype).at[g_sorted, pos_c].add(h_sorted)
    conv = jax.lax.conv_general_dilated(
        pooled, conv_w, window_strides=(1,), padding='VALID',
        dimension_numbers=('NWC', 'WIO', 'NWC'))
    conv = jax.nn.relu(conv + conv_b)
    flat = conv.reshape(N_GRAPHS, -1)
    return _head(flat, d1_w, d1_b, d2_w, d2_b)


# speedup vs baseline: 6.1068x; 1.3220x over previous
import dataclasses
import functools

import jax
import jax.numpy as jnp
from jax import lax
from jax.experimental import pallas as pl
from jax.experimental.pallas import tpu as pltpu
from jax.experimental.pallas import tpu_sc as plsc

N_NODES = 10000
N_EDGES = 320000
D = 128
N_GRAPHS = 200
K = 30
KS = 5

NC = 2
NS = 16
CHUNK = 128
CHUNKS = 80
EPT = CHUNK * CHUNKS
E_PAD = EPT * NC * NS
SBLK = 80
NSBLK = N_NODES // SBLK
DH = D // 2

_mesh = plsc.VectorSubcoreMesh(core_axis_name="c", subcore_axis_name="s")

_sc_params = pltpu.CompilerParams()
for _f, _v in (("needs_layout_passes", False), ("use_tc_tiling_on_sc", False)):
    if _f in pltpu.CompilerParams.__dataclass_fields__:
        _sc_params = dataclasses.replace(_sc_params, **{_f: _v})


def _msg_body(mp0_hbm, mp1_hbm, row_hbm, col_hbm, w_hbm, out_hbm,
              row_v, col_v, w_v, gbuf, zbuf, acc,
              gsem0, gsem1, ssem0, ssem1):
    gsem = (gsem0, gsem1)
    ssem = (ssem0, ssem1)
    c = lax.axis_index("c")
    s = lax.axis_index("s")
    pltpu.sync_copy(row_hbm.at[c, s], row_v)
    pltpu.sync_copy(col_hbm.at[c, s], col_v)
    pltpu.sync_copy(w_hbm.at[c, s], w_v)

    @pl.loop(0, SBLK)
    def _(i):
        for cc in range(DH // 16):
            zbuf[i, pl.ds(cc * 16, 16)] = jnp.zeros((16,), jnp.float32)

    for hh, mp_hbm in enumerate((mp0_hbm, mp1_hbm)):
        for k in range((NSBLK + NS - 1) // NS):
            blk = s + k * NS

            @pl.when(blk < NSBLK)
            def _():
                pltpu.sync_copy(zbuf, acc.at[pl.ds(blk * SBLK, SBLK)])
        plsc.subcore_barrier()

        for b in range(2):
            pltpu.async_copy(mp_hbm.at[row_v.at[b]], gbuf.at[b], gsem[b])

        @pl.loop(0, CHUNKS, step=2)
        def _(j):
            for b in range(2):
                jj = j + b
                gb = gbuf.at[b]
                pltpu.make_async_copy(mp_hbm.at[row_v.at[jj]], gb, gsem[b]).wait()

                @plsc.parallel_loop(0, CHUNK, unroll=4)
                def _(e):
                    wv = plsc.load_gather(
                        w_v,
                        [jnp.full((16,), jj, jnp.int32),
                         jnp.full((16,), e, jnp.int32)])
                    for cc in range(DH // 16):
                        gb[e, pl.ds(cc * 16, 16)] = gb[e, pl.ds(cc * 16, 16)] * wv

                pltpu.async_copy(gb, acc.at[col_v.at[jj]], ssem[b], add=True)

                @pl.when(jj + 2 < CHUNKS)
                def _():
                    pltpu.make_async_copy(gb, acc.at[col_v.at[jj]], ssem[b]).wait()
                    pltpu.async_copy(mp_hbm.at[row_v.at[jj + 2]], gb, gsem[b])

        for b in range(2):
            pltpu.make_async_copy(
                gbuf.at[b], acc.at[col_v.at[CHUNKS - 2 + b]], ssem[b]).wait()
        plsc.subcore_barrier()
        for k in range((NSBLK + NS - 1) // NS):
            blk = s + k * NS

            @pl.when(blk < NSBLK)
            def _():
                off = blk * SBLK
                pltpu.sync_copy(acc.at[pl.ds(off, SBLK)],
                                out_hbm.at[c, hh, pl.ds(off, SBLK)])
        plsc.subcore_barrier()


@functools.partial(
    pl.kernel,
    out_type=jax.ShapeDtypeStruct((NC, 2, N_NODES, DH), jnp.float32),
    mesh=_mesh,
    scratch_types=[
        pltpu.VMEM((CHUNKS, CHUNK), jnp.int32),
        pltpu.VMEM((CHUNKS, CHUNK), jnp.int32),
        pltpu.VMEM((CHUNKS, CHUNK), jnp.float32),
        pltpu.VMEM((2, CHUNK, DH), jnp.float32),
        pltpu.VMEM((SBLK, DH), jnp.float32),
        pltpu.VMEM_SHARED((N_NODES, DH), jnp.float32),
        pltpu.SemaphoreType.DMA,
        pltpu.SemaphoreType.DMA,
        pltpu.SemaphoreType.DMA,
        pltpu.SemaphoreType.DMA,
    ],
    compiler_params=_sc_params,
)
def _sc_msg(*args):
    _msg_body(*args)


def _head_body(flat_ref, w1_ref, b1_ref, w2_ref, b2_ref, o_ref):
    hid = jnp.maximum(flat_ref[...] @ w1_ref[...] + b1_ref[...][None, :], 0.0)
    o_ref[...] = hid @ w2_ref[...] + b2_ref[...][None, :]


def _head(flat, d1_w, d1_b, d2_w, d2_b):
    return pl.pallas_call(
        _head_body,
        out_shape=jax.ShapeDtypeStruct((flat.shape[0], d2_w.shape[1]), flat.dtype),
    )(flat, d1_w, d1_b, d2_w, d2_b)


def kernel(x, edge_index, edge_weight, node_graph_index, W1, b1, W2, b2, W3, b3,
           conv_w, conv_b, d1_w, d1_b, d2_w, d2_b):
    row = edge_index[0]
    col = edge_index[1]
    w = edge_weight
    deg = jax.ops.segment_sum(w, row, num_segments=N_NODES) + 1.0
    dis = deg ** -0.5
    dis_col = dis[:, None]
    dis2 = (dis * dis)[:, None]

    pad = E_PAD - N_EDGES
    row_a = jnp.pad(row, (0, pad)).reshape(NC, NS, CHUNKS, CHUNK)
    col_a = jnp.pad(col, (0, pad)).reshape(NC, NS, CHUNKS, CHUNK)
    w_a = jnp.pad(w, (0, pad)).reshape(NC, NS, CHUNKS, CHUNK)

    def gcn(h, W, b):
        m = h @ W
        mp = m * dis_col
        acc = _sc_msg(mp[:, :DH], mp[:, DH:], row_a, col_a, w_a)
        accsum = acc[0] + acc[1]
        agg = dis_col * jnp.concatenate([accsum[0], accsum[1]], axis=1) + dis2 * m
        return jax.nn.relu(agg + b)

    h = gcn(x, W1, b1)
    h = gcn(h, W2, b2)
    h = gcn(h, W3, b3)

    score = h[:, -1]
    order = jnp.lexsort((-score, node_graph_index))
    g_sorted = node_graph_index[order]
    counts = jnp.bincount(node_graph_index, length=N_GRAPHS)
    starts = jnp.cumsum(counts) - counts
    pos = jnp.arange(N_NODES, dtype=jnp.int32) - starts[g_sorted].astype(jnp.int32)
    mask = (pos < K).astype(h.dtype)
    h_sorted = h[order] * mask[:, None]
    pos_c = jnp.minimum(pos, K - 1)
    pooled = jnp.zeros((N_GRAPHS, K, D), dtype=h.dtype).at[g_sorted, pos_c].add(h_sorted)
    conv = jax.lax.conv_general_dilated(
        pooled, conv_w, window_strides=(1,), padding='VALID',
        dimension_numbers=('NWC', 'WIO', 'NWC'))
    conv = jax.nn.relu(conv + conv_b)
    flat = conv.reshape(N_GRAPHS, -1)
    return _head(flat, d1_w, d1_b, d2_w, d2_b)
